# Initial kernel scaffold; baseline (speedup 1.0000x reference)
import jax
import jax.numpy as jnp
from jax.experimental import pallas as pl


def kernel(skills, positions, education, job_position, skill_table, position_table, education_table, job_position_table, W1, b1, W2, b2, W3, b3):
    raise NotImplementedError("write your pallas kernel here")


# speedup vs baseline: 1.7899x; 1.7899x over previous
import functools

import jax
import jax.numpy as jnp
from jax import lax
from jax.experimental import pallas as pl
from jax.experimental.pallas import tpu as pltpu
from jax.experimental.pallas import tpu_sc as plsc

B = 16384
D = 16
NC = 2
NS = 16
NW = NC * NS
BPW = B // NW
CHUNK = 128
NCHUNK = BPW // CHUNK

_mesh = plsc.VectorSubcoreMesh(core_axis_name="c", subcore_axis_name="s")


@functools.partial(
    pl.kernel,
    out_type=[jax.ShapeDtypeStruct((B, D), jnp.float32) for _ in range(4)],
    mesh=_mesh,
    scratch_types=[
        [pltpu.VMEM((NCHUNK, CHUNK), jnp.int32) for _ in range(4)],
        [pltpu.VMEM((BPW, D), jnp.float32) for _ in range(4)],
        pltpu.SemaphoreType.DMA,
        pltpu.SemaphoreType.DMA,
    ],
    compiler_params=pltpu.CompilerParams(use_tc_tiling_on_sc=False),
)
def _gather4(sk_i, po_i, ed_i, jp_i,
             sk_t, po_t, ed_t, jp_t,
             sk_o, po_o, ed_o, jp_o,
             idx_v, rows_v, sem_idx, sem_rows):
    wid = lax.axis_index("s") * NC + lax.axis_index("c")
    base = wid * BPW

    idx_hbm = (sk_i, po_i, ed_i, jp_i)
    tables = (sk_t, po_t, ed_t, jp_t)
    outs = (sk_o, po_o, ed_o, jp_o)

    idx_copies = [
        pltpu.make_async_copy(
            idx_hbm[t].at[pl.ds(base + j * CHUNK, CHUNK)],
            idx_v[t].at[j],
            sem_idx,
        )
        for t in range(4)
        for j in range(NCHUNK)
    ]
    for c in idx_copies:
        c.start()
    for c in idx_copies:
        c.wait()

    row_copies = [
        pltpu.make_async_copy(
            tables[t].at[idx_v[t].at[j]],
            rows_v[t].at[pl.ds(j * CHUNK, CHUNK)],
            sem_rows,
        )
        for t in range(4)
        for j in range(NCHUNK)
    ]
    for c in row_copies:
        c.start()
    for c in row_copies:
        c.wait()

    out_copies = [
        pltpu.make_async_copy(
            rows_v[t].at[...],
            outs[t].at[pl.ds(base, BPW)],
            sem_idx,
        )
        for t in range(4)
    ]
    for c in out_copies:
        c.start()
    for c in out_copies:
        c.wait()


BLK = 2048


def _mlp_body(s_ref, p_ref, e_ref, j_ref,
              w1_ref, b1_ref, w2_ref, b2_ref, w3_ref, b3_ref, o_ref):
    w1 = w1_ref[...]
    h = (
        jnp.dot(s_ref[...], w1[0:D, :], preferred_element_type=jnp.float32)
        + jnp.dot(p_ref[...], w1[D:2 * D, :], preferred_element_type=jnp.float32)
        + jnp.dot(e_ref[...], w1[2 * D:3 * D, :], preferred_element_type=jnp.float32)
        + jnp.dot(j_ref[...], w1[3 * D:4 * D, :], preferred_element_type=jnp.float32)
        + b1_ref[...]
    )
    h = jnp.maximum(h, 0.0)
    h = jnp.dot(h, w2_ref[...], preferred_element_type=jnp.float32) + b2_ref[...]
    h = jnp.maximum(h, 0.0)
    z = jnp.dot(h, w3_ref[...], preferred_element_type=jnp.float32) + b3_ref[...]
    o_ref[...] = jax.nn.sigmoid(z)


def _mlp(s, p, e, j, W1, b1, W2, b2, W3, b3):
    grid = (B // BLK,)
    emb_spec = pl.BlockSpec((BLK, D), lambda i: (i, 0))
    whole = lambda shape: pl.BlockSpec(shape, lambda i: (0,) * len(shape))
    return pl.pallas_call(
        _mlp_body,
        grid=grid,
        in_specs=[
            emb_spec, emb_spec, emb_spec, emb_spec,
            whole((4 * D, 64)), whole((1, 64)),
            whole((64, 32)), whole((1, 32)),
            whole((32, 1)), whole((1, 1)),
        ],
        out_specs=pl.BlockSpec((BLK, 1), lambda i: (i, 0)),
        out_shape=jax.ShapeDtypeStruct((B, 1), jnp.float32),
    )(s, p, e, j, W1, b1, W2, b2, W3, b3)


def kernel(skills, positions, education, job_position,
           skill_table, position_table, education_table, job_position_table,
           W1, b1, W2, b2, W3, b3):
    sk = skills.astype(jnp.int32)
    po = positions.astype(jnp.int32)
    ed = education.astype(jnp.int32)
    jp = job_position.astype(jnp.int32)

    s_emb, p_emb, e_emb, j_emb = _gather4(
        sk, po, ed, jp,
        skill_table, position_table, education_table, job_position_table,
    )

    out = _mlp(
        s_emb, p_emb, e_emb, j_emb,
        W1, b1.reshape(1, 64), W2, b2.reshape(1, 32), W3, b3.reshape(1, 1),
    )
    return jnp.squeeze(out, axis=-1)


# speedup vs baseline: 2.1126x; 1.1803x over previous
import functools

import jax
import jax.numpy as jnp
from jax import lax
from jax.experimental import pallas as pl
from jax.experimental.pallas import tpu as pltpu
from jax.experimental.pallas import tpu_sc as plsc

B = 16384
D = 16
NC = 2
NS = 16
NW = NC * NS
BPW = B // NW
HALF = BPW // 2
CHUNK = 128
NCHUNK = BPW // CHUNK

_mesh = plsc.VectorSubcoreMesh(core_axis_name="c", subcore_axis_name="s")


@functools.partial(
    pl.kernel,
    out_type=jax.ShapeDtypeStruct((B // 2, 8 * D), jnp.float32),
    mesh=_mesh,
    scratch_types=[
        [pltpu.VMEM((NCHUNK, CHUNK), jnp.int32) for _ in range(4)],
        [pltpu.VMEM((BPW, D), jnp.float32) for _ in range(4)],
        pltpu.SemaphoreType.DMA,
        pltpu.SemaphoreType.DMA,
    ],
    compiler_params=pltpu.CompilerParams(use_tc_tiling_on_sc=False),
)
def _gather4(sk_i, po_i, ed_i, jp_i,
             sk_t, po_t, ed_t, jp_t,
             x_o, idx_v, rows_v, sem_idx, sem_rows):
    wid = lax.axis_index("s") * NC + lax.axis_index("c")
    base = wid * BPW

    idx_hbm = (sk_i, po_i, ed_i, jp_i)
    tables = (sk_t, po_t, ed_t, jp_t)

    idx_copies = [
        pltpu.make_async_copy(
            idx_hbm[t].at[pl.ds(base + j * CHUNK, CHUNK)],
            idx_v[t].at[j],
            sem_idx,
        )
        for t in range(4)
        for j in range(NCHUNK)
    ]
    for c in idx_copies:
        c.start()
    for c in idx_copies:
        c.wait()

    row_copies = [
        pltpu.make_async_copy(
            tables[t].at[idx_v[t].at[j]],
            rows_v[t].at[pl.ds(j * CHUNK, CHUNK)],
            sem_rows,
        )
        for t in range(4)
        for j in range(NCHUNK)
    ]
    for c in row_copies:
        c.start()
    for c in row_copies:
        c.wait()

    out_copies = [
        pltpu.make_async_copy(
            rows_v[t].at[pl.ds(h * HALF, HALF)],
            x_o.at[pl.ds(wid * HALF, HALF), pl.ds((4 * h + t) * D, D)],
            sem_idx,
        )
        for t in range(4)
        for h in range(2)
    ]
    for c in out_copies:
        c.start()
    for c in out_copies:
        c.wait()


BLK = 2048


def _mlp_body(x_ref, w1_ref, b1_ref, w2_ref, b2_ref,
              w3e_ref, w3o_ref, b3_ref, oe_ref, oo_ref):
    h = jnp.dot(x_ref[...], w1_ref[...], preferred_element_type=jnp.float32)
    h = jnp.maximum(h + b1_ref[...], 0.0)
    h2 = jnp.dot(h, w2_ref[...], preferred_element_type=jnp.float32)
    h2 = jnp.maximum(h2 + b2_ref[...], 0.0)
    b3 = b3_ref[0, 0]
    ze = jnp.sum(h2 * w3e_ref[...], axis=1) + b3
    zo = jnp.sum(h2 * w3o_ref[...], axis=1) + b3
    oe_ref[...] = jax.nn.sigmoid(ze)
    oo_ref[...] = jax.nn.sigmoid(zo)


def _mlp(x2, W1bd, b1bd, W2bd, b2bd, w3e, w3o, b3):
    grid = ((B // 2) // BLK,)
    whole = lambda shape: pl.BlockSpec(shape, lambda i: (0,) * len(shape))
    return pl.pallas_call(
        _mlp_body,
        grid=grid,
        in_specs=[
            pl.BlockSpec((BLK, 128), lambda i: (i, 0)),
            whole((128, 128)), whole((1, 128)),
            whole((128, 64)), whole((1, 64)),
            whole((1, 64)), whole((1, 64)), whole((1, 1)),
        ],
        out_specs=[
            pl.BlockSpec((BLK,), lambda i: (i,)),
            pl.BlockSpec((BLK,), lambda i: (i,)),
        ],
        out_shape=[
            jax.ShapeDtypeStruct((B // 2,), jnp.float32),
            jax.ShapeDtypeStruct((B // 2,), jnp.float32),
        ],
    )(x2, W1bd, b1bd, W2bd, b2bd, w3e, w3o, b3)


def kernel(skills, positions, education, job_position,
           skill_table, position_table, education_table, job_position_table,
           W1, b1, W2, b2, W3, b3):
    sk = skills.astype(jnp.int32)
    po = positions.astype(jnp.int32)
    ed = education.astype(jnp.int32)
    jp = job_position.astype(jnp.int32)

    x2 = _gather4(
        sk, po, ed, jp,
        skill_table, position_table, education_table, job_position_table,
    )

    z64 = jnp.zeros((64, 64), jnp.float32)
    z32 = jnp.zeros((64, 32), jnp.float32)
    W1bd = jnp.block([[W1, z64], [z64, W1]])
    W2bd = jnp.block([[W2, z32], [z32, W2]])
    b1bd = jnp.concatenate([b1, b1]).reshape(1, 128)
    b2bd = jnp.concatenate([b2, b2]).reshape(1, 64)
    w3 = W3[:, 0]
    zv = jnp.zeros((32,), jnp.float32)
    w3e = jnp.concatenate([w3, zv]).reshape(1, 64)
    w3o = jnp.concatenate([zv, w3]).reshape(1, 64)

    oe, oo = _mlp(x2, W1bd, b1bd, W2bd, b2bd, w3e, w3o, b3.reshape(1, 1))

    return jnp.concatenate(
        [oe.reshape(NW, HALF), oo.reshape(NW, HALF)], axis=1
    ).reshape(B)


# speedup vs baseline: 3.9508x; 1.8701x over previous
import functools

import jax
import jax.numpy as jnp
from jax import lax
from jax.experimental import pallas as pl
from jax.experimental.pallas import tpu as pltpu
from jax.experimental.pallas import tpu_sc as plsc

B = 16384
D = 16
V = 100000
VE = 1000
NC = 2
NS = 16
NW = NC * NS
L = 16
HB = B // 2

_mesh = plsc.VectorSubcoreMesh(core_axis_name="c", subcore_axis_name="s")


@functools.partial(
    pl.kernel,
    out_type=jax.ShapeDtypeStruct((4 * D, B), jnp.float32),
    mesh=_mesh,
    scratch_types=[
        pltpu.VMEM((V,), jnp.float32),
        pltpu.VMEM((HB,), jnp.int32),
        pltpu.VMEM((B,), jnp.float32),
        pltpu.SemaphoreType.DMA,
        pltpu.SemaphoreType.DMA,
    ],
    compiler_params=pltpu.CompilerParams(
        use_tc_tiling_on_sc=False, needs_layout_passes=False
    ),
)
def _gatherT(sk_i, po_i, ed_i, jp_i,
             sk_t, po_t, ed_t, jp_t,
             xT, col_v, idx_v, out_v, sem_a, sem_b):
    wid = lax.axis_index("s") * NC + lax.axis_index("c")
    t = lax.shift_right_logical(wid, 3)
    f0 = (2 * wid) & 15

    idx_hbm = (sk_i, po_i, ed_i, jp_i)
    tables = (sk_t, po_t, ed_t, jp_t)
    sizes = (V, V, VE, V)

    def _row(k, ri):
        f = f0 + ri
        r = 16 * k + f
        cp = pltpu.make_async_copy(
            tables[k].at[f, pl.ds(0, sizes[k])],
            col_v.at[pl.ds(0, sizes[k])],
            sem_a,
        )
        cp.start()
        cp.wait()
        for h in range(2):
            ci = pltpu.make_async_copy(
                idx_hbm[k].at[pl.ds(h * HB, HB)], idx_v.at[...], sem_b
            )
            ci.start()
            ci.wait()

            def _grp(g, _):
                iv = idx_v[pl.ds(g * L, L)]
                out_v[pl.ds(h * HB + g * L, L)] = plsc.load_gather(col_v, [iv])
                return 0

            lax.fori_loop(0, HB // L, _grp, 0)
        co = pltpu.make_async_copy(out_v.at[...], xT.at[r], sem_a)
        co.start()
        co.wait()

    for k in range(4):
        @pl.when(t == k)
        def _():
            _row(k, 0)
            _row(k, 1)


BLK = 4096


def _mlp_body(x_ref, w1_ref, b1_ref, w2_ref, b2_ref, w3_ref, b3_ref, o_ref):
    dn = (((0,), (0,)), ((), ()))
    h = lax.dot_general(w1_ref[...], x_ref[...], dn,
                        preferred_element_type=jnp.float32)
    h = jnp.maximum(h + b1_ref[...], 0.0)
    h2 = lax.dot_general(w2_ref[...], h, dn,
                         preferred_element_type=jnp.float32)
    h2 = jnp.maximum(h2 + b2_ref[...], 0.0)
    z = lax.dot_general(w3_ref[...], h2, dn,
                        preferred_element_type=jnp.float32)
    o_ref[...] = jax.nn.sigmoid(z + b3_ref[0, 0])[0]


def _mlp(xT, W1, b1, W2, b2, W3, b3):
    grid = (B // BLK,)
    whole = lambda shape: pl.BlockSpec(shape, lambda i: (0,) * len(shape))
    return pl.pallas_call(
        _mlp_body,
        grid=grid,
        in_specs=[
            pl.BlockSpec((64, BLK), lambda i: (0, i)),
            whole((64, 64)), whole((64, 1)),
            whole((64, 32)), whole((32, 1)),
            whole((32, 1)), whole((1, 1)),
        ],
        out_specs=pl.BlockSpec((BLK,), lambda i: (i,)),
        out_shape=jax.ShapeDtypeStruct((B,), jnp.float32),
    )(xT, W1, b1, W2, b2, W3, b3)


def kernel(skills, positions, education, job_position,
           skill_table, position_table, education_table, job_position_table,
           W1, b1, W2, b2, W3, b3):
    sk = skills.astype(jnp.int32)
    po = positions.astype(jnp.int32)
    ed = education.astype(jnp.int32)
    jp = job_position.astype(jnp.int32)

    xT = _gatherT(
        sk, po, ed, jp,
        skill_table.T, position_table.T, education_table.T,
        job_position_table.T,
    )

    return _mlp(
        xT, W1, b1.reshape(64, 1), W2, b2.reshape(32, 1),
        W3, b3.reshape(1, 1),
    )
